# baseline (speedup 1.0000x reference)
import functools

import jax
import jax.numpy as jnp
from jax import lax
from jax.experimental import pallas as pl
from jax.experimental.pallas import tpu as pltpu
from jax.experimental.pallas import tpu_sc as plsc

B = 2048
D = 1024
E = 8
K = 2
H = 4096
C = 1024

BLK = 512
NBLK = (B * K) // BLK + E
NPAD = NBLK * BLK
NUSED_SLOT = 32

RBLK = 256
NEG_INF = float("-inf")


def _router_body(x_ref, wgt_ref, bg_ref,
                 rout_ref, topk_ref, pos_ref, w_ref, emap_ref, xpk_ref):
    x = x_ref[...]
    logits = lax.dot_general(x, wgt_ref[...], (((1,), (1,)), ((), ())),
                             preferred_element_type=jnp.float32) + bg_ref[...]

    iota_e = lax.broadcasted_iota(jnp.int32, (B, E), 1)
    v0 = jnp.max(logits, axis=1, keepdims=True)
    i0 = jnp.min(jnp.where(logits == v0, iota_e, E), axis=1, keepdims=True)
    oh0 = iota_e == i0
    masked = jnp.where(oh0, NEG_INF, logits)
    v1 = jnp.max(masked, axis=1, keepdims=True)
    i1 = jnp.min(jnp.where(masked == v1, iota_e, E), axis=1, keepdims=True)
    oh1 = iota_e == i1

    e1 = jnp.exp(v1 - v0)
    w0 = 1.0 / (1.0 + e1)
    w1 = e1 * w0
    rout_ref[...] = jnp.where(oh0, w0, 0.0) + jnp.where(oh1, w1, 0.0)
    topk_ref[...] = jnp.concatenate([i0, i1], axis=1)

    combined = oh0.astype(jnp.float32) + oh1.astype(jnp.float32)
    nblocks = B // RBLK
    base = jnp.zeros((1, E), dtype=jnp.float32)
    rank_parts = []
    tri = (lax.broadcasted_iota(jnp.int32, (RBLK, RBLK), 1)
           < lax.broadcasted_iota(jnp.int32, (RBLK, RBLK), 0)).astype(jnp.float32)
    for blk in range(nblocks):
        rows = combined[blk * RBLK:(blk + 1) * RBLK]
        rk = jnp.dot(tri, rows, preferred_element_type=jnp.float32) + base
        rank_parts.append(rk)
        base = base + jnp.sum(rows, axis=0, keepdims=True)
    rank = jnp.concatenate(rank_parts, axis=0)
    counts = base

    padded = jnp.ceil(counts / BLK) * BLK
    lt8 = (lax.broadcasted_iota(jnp.int32, (E, E), 0)
           < lax.broadcasted_iota(jnp.int32, (E, E), 1)).astype(jnp.float32)
    offs = jnp.dot(padded, lt8, preferred_element_type=jnp.float32)

    posmat = offs + rank
    pos0 = jnp.sum(jnp.where(oh0, posmat, 0.0), axis=1, keepdims=True)
    pos1 = jnp.sum(jnp.where(oh1, posmat, 0.0), axis=1, keepdims=True)
    pos_ref[0:1, :] = pos0.astype(jnp.int32).reshape(1, B)
    pos_ref[1:2, :] = pos1.astype(jnp.int32).reshape(1, B)
    w_ref[...] = jnp.concatenate([w0, w1], axis=1)

    bstart = (offs / BLK).astype(jnp.int32).reshape(E, 1)
    bidx = lax.broadcasted_iota(jnp.int32, (E, 64), 1)
    emap = (jnp.sum((bstart <= bidx).astype(jnp.int32), axis=0,
                    keepdims=True) - 1)
    nused = (jnp.sum(padded) / BLK).astype(jnp.int32)
    emap_ref[...] = jnp.where(
        lax.broadcasted_iota(jnp.int32, (1, 64), 1) == NUSED_SLOT,
        nused, emap)

    xeb = pltpu.bitcast(x[:, :D // 2].astype(jnp.bfloat16),
                        jnp.int16).astype(jnp.int32)
    xob = pltpu.bitcast(x[:, D // 2:].astype(jnp.bfloat16),
                        jnp.int16).astype(jnp.int32)
    xpk_ref[...] = (xob << 16) | (xeb & 0xFFFF)


def _router(x, wgt, bg2):
    return pl.pallas_call(
        _router_body,
        out_shape=(
            jax.ShapeDtypeStruct((B, E), jnp.float32),
            jax.ShapeDtypeStruct((B, K), jnp.int32),
            jax.ShapeDtypeStruct((K, B), jnp.int32),
            jax.ShapeDtypeStruct((B, K), jnp.float32),
            jax.ShapeDtypeStruct((1, 64), jnp.int32),
            jax.ShapeDtypeStruct((B, D // 2), jnp.int32),
        ),
    )(x, wgt, bg2)


NW = 32
TPW = B // NW


def _scatter_body(x_hbm, pos_hbm, xg_hbm, rows_v, idx0_v, idx1_v, sem0, sem1):
    wid = lax.axis_index("s") * 2 + lax.axis_index("c")
    base = wid * TPW
    pltpu.sync_copy(x_hbm.at[pl.ds(base, TPW)], rows_v)
    pltpu.sync_copy(pos_hbm.at[0, pl.ds(base, TPW)], idx0_v)
    pltpu.sync_copy(pos_hbm.at[1, pl.ds(base, TPW)], idx1_v)
    cp0 = pltpu.make_async_copy(rows_v, xg_hbm.at[idx0_v], sem0)
    cp1 = pltpu.make_async_copy(rows_v, xg_hbm.at[idx1_v], sem1)
    cp0.start()
    cp1.start()
    cp0.wait()
    cp1.wait()


def _scatter(xpk, pos):
    f = pl.kernel(
        _scatter_body,
        out_type=jax.ShapeDtypeStruct((NPAD, D // 2), jnp.int32),
        mesh=plsc.VectorSubcoreMesh(core_axis_name="c", subcore_axis_name="s"),
        scratch_types=[
            pltpu.VMEM((TPW, D // 2), jnp.int32),
            pltpu.VMEM((TPW,), jnp.int32),
            pltpu.VMEM((TPW,), jnp.int32),
            pltpu.SemaphoreType.DMA,
            pltpu.SemaphoreType.DMA,
        ],
    )
    return f(xpk, pos)


_SQRT_HALF = 0.7071067811865476


def _mm_body(emap_ref, xg_ref, w1_ref, b1_ref, w2_ref, b2_ref, yg_ref,
             acc_ref):
    i = pl.program_id(0)
    j = pl.program_id(1)

    @pl.when(i < emap_ref[0, NUSED_SLOT])
    def _():
        xp = xg_ref[...]
        lo = pltpu.bitcast((xp & 0xFFFF) << 16, jnp.float32)
        hi = pltpu.bitcast(xp & -65536, jnp.float32)
        xb = jnp.concatenate([lo, hi], axis=1)
        pre = jnp.dot(xb, w1_ref[0],
                      preferred_element_type=jnp.float32) + b1_ref[0]
        hj = 0.5 * pre * (1.0 + lax.erf(pre * _SQRT_HALF))
        contrib = jnp.dot(hj, w2_ref[0],
                          preferred_element_type=jnp.float32)

        @pl.when(j == 0)
        def _():
            acc_ref[...] = contrib

        @pl.when(j == 1)
        def _():
            yg = acc_ref[...] + contrib + b2_ref[0]
            ye = pltpu.bitcast(yg[:, :C // 2].astype(jnp.bfloat16),
                               jnp.int16).astype(jnp.int32)
            yo = pltpu.bitcast(yg[:, C // 2:].astype(jnp.bfloat16),
                               jnp.int16).astype(jnp.int32)
            yg_ref[...] = (yo << 16) | (ye & 0xFFFF)


def _clamp(i, em):
    return jnp.minimum(i, em[0, NUSED_SLOT] - 1)


def _moe_mm(emap, xg, W1, b1, W2, b2):
    return pl.pallas_call(
        _mm_body,
        grid_spec=pltpu.PrefetchScalarGridSpec(
            num_scalar_prefetch=1,
            grid=(NBLK, 2),
            in_specs=[
                pl.BlockSpec((BLK, D // 2), lambda i, j, em: (_clamp(i, em), 0)),
                pl.BlockSpec((1, D, H // 2),
                             lambda i, j, em: (em[0, _clamp(i, em)], 0, j)),
                pl.BlockSpec((1, 1, H // 2),
                             lambda i, j, em: (em[0, _clamp(i, em)], 0, j)),
                pl.BlockSpec((1, H // 2, C),
                             lambda i, j, em: (em[0, _clamp(i, em)], j, 0)),
                pl.BlockSpec((1, 1, C),
                             lambda i, j, em: (em[0, _clamp(i, em)], 0, 0)),
            ],
            out_specs=pl.BlockSpec((BLK, C // 2),
                                   lambda i, j, em: (_clamp(i, em), 0)),
            scratch_shapes=[pltpu.VMEM((BLK, C), jnp.float32)],
        ),
        out_shape=jax.ShapeDtypeStruct((NPAD, C // 2), jnp.int32),
    )(emap, xg, W1, b1.reshape(E, 1, H), W2, b2.reshape(E, 1, C))


CH = 32


def _gather2_body(yg_hbm, pos_hbm, g_hbm,
                  r0_v, r1_v, idx0_v, idx1_v, sem0, sem1):
    wid = lax.axis_index("s") * 2 + lax.axis_index("c")
    base = wid * TPW
    pltpu.sync_copy(pos_hbm.at[0, pl.ds(base, TPW)], idx0_v)
    pltpu.sync_copy(pos_hbm.at[1, pl.ds(base, TPW)], idx1_v)
    cp0 = pltpu.make_async_copy(yg_hbm.at[idx0_v], r0_v, sem0)
    cp1 = pltpu.make_async_copy(yg_hbm.at[idx1_v], r1_v, sem1)
    cp0.start()
    cp1.start()
    cp0.wait()
    pltpu.sync_copy(r0_v, g_hbm.at[0, pl.ds(base, TPW)])
    cp1.wait()
    pltpu.sync_copy(r1_v, g_hbm.at[1, pl.ds(base, TPW)])


def _gather2(yg, pos):
    f = pl.kernel(
        _gather2_body,
        out_type=jax.ShapeDtypeStruct((K, B, C // 2), jnp.int32),
        mesh=plsc.VectorSubcoreMesh(core_axis_name="c", subcore_axis_name="s"),
        scratch_types=[
            pltpu.VMEM((TPW, C // 2), jnp.int32),
            pltpu.VMEM((TPW, C // 2), jnp.int32),
            pltpu.VMEM((TPW,), jnp.int32),
            pltpu.VMEM((TPW,), jnp.int32),
            pltpu.SemaphoreType.DMA,
            pltpu.SemaphoreType.DMA,
        ],
    )
    return f(yg, pos)


FRB = 512


def _finalize_body(g_ref, w_ref, out_ref):
    g0 = g_ref[0]
    g1 = g_ref[1]
    w0c = w_ref[:, 0:1]
    w1c = w_ref[:, 1:2]
    lo0 = pltpu.bitcast((g0 & 0xFFFF) << 16, jnp.float32)
    hi0 = pltpu.bitcast(g0 & -65536, jnp.float32)
    lo1 = pltpu.bitcast((g1 & 0xFFFF) << 16, jnp.float32)
    hi1 = pltpu.bitcast(g1 & -65536, jnp.float32)
    out_ref[:, 0:C // 2] = w0c * lo0 + w1c * lo1
    out_ref[:, C // 2:C] = w0c * hi0 + w1c * hi1


def _finalize(g, wcols):
    return pl.pallas_call(
        _finalize_body,
        grid=(B // FRB,),
        in_specs=[
            pl.BlockSpec((K, FRB, C // 2), lambda i: (0, i, 0)),
            pl.BlockSpec((FRB, K), lambda i: (i, 0)),
        ],
        out_specs=pl.BlockSpec((FRB, C), lambda i: (i, 0)),
        out_shape=jax.ShapeDtypeStruct((B, C), jnp.float32),
    )(g, wcols)


def kernel(x, Wg, bg, W1, b1, W2, b2):
    rout, topk, pos, wcols, emap_row, xpk = _router(x, Wg, bg.reshape(1, E))
    xg = _scatter(xpk, pos)
    yg = _moe_mm(emap_row, xg, W1, b1, W2, b2)
    g = _gather2(yg, pos)
    final = _finalize(g, wcols)
    return final, rout, topk


# speedup vs baseline: 1.0732x; 1.0732x over previous
import functools

import jax
import jax.numpy as jnp
from jax import lax
from jax.experimental import pallas as pl
from jax.experimental.pallas import tpu as pltpu
from jax.experimental.pallas import tpu_sc as plsc

B = 2048
D = 1024
E = 8
K = 2
H = 4096
C = 1024

BLK = 512
NBLK = (B * K) // BLK + E
NPAD = NBLK * BLK
NUSED_SLOT = 32

RBLK = 256
NEG_INF = float("-inf")


def _router_body(x_ref, wgt_ref, bg_ref,
                 rout_ref, topk_ref, pos_ref, w_ref, emap_ref, xpk_ref):
    x = x_ref[...]
    logits = lax.dot_general(x, wgt_ref[...], (((1,), (1,)), ((), ())),
                             preferred_element_type=jnp.float32) + bg_ref[...]

    iota_e = lax.broadcasted_iota(jnp.int32, (B, E), 1)
    v0 = jnp.max(logits, axis=1, keepdims=True)
    i0 = jnp.min(jnp.where(logits == v0, iota_e, E), axis=1, keepdims=True)
    oh0 = iota_e == i0
    masked = jnp.where(oh0, NEG_INF, logits)
    v1 = jnp.max(masked, axis=1, keepdims=True)
    i1 = jnp.min(jnp.where(masked == v1, iota_e, E), axis=1, keepdims=True)
    oh1 = iota_e == i1

    e1 = jnp.exp(v1 - v0)
    w0 = 1.0 / (1.0 + e1)
    w1 = e1 * w0
    rout_ref[...] = jnp.where(oh0, w0, 0.0) + jnp.where(oh1, w1, 0.0)
    topk_ref[...] = jnp.concatenate([i0, i1], axis=1)

    combined = oh0.astype(jnp.float32) + oh1.astype(jnp.float32)
    nblocks = B // RBLK
    base = jnp.zeros((1, E), dtype=jnp.float32)
    rank_parts = []
    tri = (lax.broadcasted_iota(jnp.int32, (RBLK, RBLK), 1)
           < lax.broadcasted_iota(jnp.int32, (RBLK, RBLK), 0)).astype(jnp.float32)
    for blk in range(nblocks):
        rows = combined[blk * RBLK:(blk + 1) * RBLK]
        rk = jnp.dot(tri, rows, preferred_element_type=jnp.float32) + base
        rank_parts.append(rk)
        base = base + jnp.sum(rows, axis=0, keepdims=True)
    rank = jnp.concatenate(rank_parts, axis=0)
    counts = base

    padded = jnp.ceil(counts / BLK) * BLK
    lt8 = (lax.broadcasted_iota(jnp.int32, (E, E), 0)
           < lax.broadcasted_iota(jnp.int32, (E, E), 1)).astype(jnp.float32)
    offs = jnp.dot(padded, lt8, preferred_element_type=jnp.float32)

    posmat = offs + rank
    pos0 = jnp.sum(jnp.where(oh0, posmat, 0.0), axis=1, keepdims=True)
    pos1 = jnp.sum(jnp.where(oh1, posmat, 0.0), axis=1, keepdims=True)
    pos_ref[0:1, :] = pos0.astype(jnp.int32).reshape(1, B)
    pos_ref[1:2, :] = pos1.astype(jnp.int32).reshape(1, B)
    w_ref[...] = jnp.concatenate([w0, w1], axis=1)

    bstart = (offs / BLK).astype(jnp.int32).reshape(E, 1)
    bidx = lax.broadcasted_iota(jnp.int32, (E, 64), 1)
    emap = (jnp.sum((bstart <= bidx).astype(jnp.int32), axis=0,
                    keepdims=True) - 1)
    nused = (jnp.sum(padded) / BLK).astype(jnp.int32)
    emap_ref[...] = jnp.where(
        lax.broadcasted_iota(jnp.int32, (1, 64), 1) == NUSED_SLOT,
        nused, emap)

    xeb = pltpu.bitcast(x[:, :D // 2].astype(jnp.bfloat16),
                        jnp.int16).astype(jnp.int32)
    xob = pltpu.bitcast(x[:, D // 2:].astype(jnp.bfloat16),
                        jnp.int16).astype(jnp.int32)
    xpk_ref[...] = (xob << 16) | (xeb & 0xFFFF)


def _router(x, wgt, bg2):
    return pl.pallas_call(
        _router_body,
        out_shape=(
            jax.ShapeDtypeStruct((B, E), jnp.float32),
            jax.ShapeDtypeStruct((B, K), jnp.int32),
            jax.ShapeDtypeStruct((K, B), jnp.int32),
            jax.ShapeDtypeStruct((B, K), jnp.float32),
            jax.ShapeDtypeStruct((1, 64), jnp.int32),
            jax.ShapeDtypeStruct((B, D // 2), jnp.int32),
        ),
    )(x, wgt, bg2)


NW = 32
TPW = B // NW


def _scatter_body(x_hbm, pos_hbm, xg_hbm, rows_v, idx0_v, idx1_v, sem0, sem1):
    wid = lax.axis_index("s") * 2 + lax.axis_index("c")
    base = wid * TPW
    pltpu.sync_copy(x_hbm.at[pl.ds(base, TPW)], rows_v)
    pltpu.sync_copy(pos_hbm.at[0, pl.ds(base, TPW)], idx0_v)
    pltpu.sync_copy(pos_hbm.at[1, pl.ds(base, TPW)], idx1_v)
    cp0 = pltpu.make_async_copy(rows_v, xg_hbm.at[idx0_v], sem0)
    cp1 = pltpu.make_async_copy(rows_v, xg_hbm.at[idx1_v], sem1)
    cp0.start()
    cp1.start()
    cp0.wait()
    cp1.wait()


def _scatter(xpk, pos):
    f = pl.kernel(
        _scatter_body,
        out_type=jax.ShapeDtypeStruct((NPAD, D // 2), jnp.int32),
        mesh=plsc.VectorSubcoreMesh(core_axis_name="c", subcore_axis_name="s"),
        scratch_types=[
            pltpu.VMEM((TPW, D // 2), jnp.int32),
            pltpu.VMEM((TPW,), jnp.int32),
            pltpu.VMEM((TPW,), jnp.int32),
            pltpu.SemaphoreType.DMA,
            pltpu.SemaphoreType.DMA,
        ],
    )
    return f(xpk, pos)


_SQRT_HALF = 0.7071067811865476


def _mm_body(emap_ref, xg_ref, w1_ref, b1_ref, w2_ref, b2_ref, yg_ref,
             acc_ref):
    i = pl.program_id(0)
    j = pl.program_id(1)

    @pl.when(i < emap_ref[0, NUSED_SLOT])
    def _():
        xp = xg_ref[...]
        lo = pltpu.bitcast((xp & 0xFFFF) << 16, jnp.float32)
        hi = pltpu.bitcast(xp & -65536, jnp.float32)
        xb = jnp.concatenate([lo, hi], axis=1)
        pre = jnp.dot(xb, w1_ref[0],
                      preferred_element_type=jnp.float32) + b1_ref[0]
        hj = 0.5 * pre * (1.0 + lax.erf(pre * _SQRT_HALF))
        contrib = jnp.dot(hj, w2_ref[0],
                          preferred_element_type=jnp.float32)

        @pl.when(j == 0)
        def _():
            acc_ref[...] = contrib

        @pl.when(j == 1)
        def _():
            yg = acc_ref[...] + contrib + b2_ref[0]
            ye = pltpu.bitcast(yg[:, :C // 2].astype(jnp.bfloat16),
                               jnp.int16).astype(jnp.int32)
            yo = pltpu.bitcast(yg[:, C // 2:].astype(jnp.bfloat16),
                               jnp.int16).astype(jnp.int32)
            yg_ref[...] = (yo << 16) | (ye & 0xFFFF)


def _clamp(i, em):
    return jnp.minimum(i, em[0, NUSED_SLOT] - 1)


def _moe_mm(emap, xg, W1, b1, W2, b2):
    return pl.pallas_call(
        _mm_body,
        grid_spec=pltpu.PrefetchScalarGridSpec(
            num_scalar_prefetch=1,
            grid=(NBLK, 2),
            in_specs=[
                pl.BlockSpec((BLK, D // 2), lambda i, j, em: (_clamp(i, em), 0)),
                pl.BlockSpec((1, D, H // 2),
                             lambda i, j, em: (em[0, _clamp(i, em)], 0,
                                               (i + j) % 2)),
                pl.BlockSpec((1, 1, H // 2),
                             lambda i, j, em: (em[0, _clamp(i, em)], 0,
                                               (i + j) % 2)),
                pl.BlockSpec((1, H // 2, C),
                             lambda i, j, em: (em[0, _clamp(i, em)],
                                               (i + j) % 2, 0)),
                pl.BlockSpec((1, 1, C),
                             lambda i, j, em: (em[0, _clamp(i, em)], 0, 0)),
            ],
            out_specs=pl.BlockSpec((BLK, C // 2),
                                   lambda i, j, em: (_clamp(i, em), 0)),
            scratch_shapes=[pltpu.VMEM((BLK, C), jnp.float32)],
        ),
        out_shape=jax.ShapeDtypeStruct((NPAD, C // 2), jnp.int32),
    )(emap, xg, W1, b1.reshape(E, 1, H), W2, b2.reshape(E, 1, C))


CH = 32


def _gather2_body(yg_hbm, pos_hbm, g_hbm,
                  r0_v, r1_v, idx0_v, idx1_v, sem0, sem1):
    wid = lax.axis_index("s") * 2 + lax.axis_index("c")
    base = wid * TPW
    pltpu.sync_copy(pos_hbm.at[0, pl.ds(base, TPW)], idx0_v)
    pltpu.sync_copy(pos_hbm.at[1, pl.ds(base, TPW)], idx1_v)
    cp0 = pltpu.make_async_copy(yg_hbm.at[idx0_v], r0_v, sem0)
    cp1 = pltpu.make_async_copy(yg_hbm.at[idx1_v], r1_v, sem1)
    cp0.start()
    cp1.start()
    cp0.wait()
    pltpu.sync_copy(r0_v, g_hbm.at[0, pl.ds(base, TPW)])
    cp1.wait()
    pltpu.sync_copy(r1_v, g_hbm.at[1, pl.ds(base, TPW)])


def _gather2(yg, pos):
    f = pl.kernel(
        _gather2_body,
        out_type=jax.ShapeDtypeStruct((K, B, C // 2), jnp.int32),
        mesh=plsc.VectorSubcoreMesh(core_axis_name="c", subcore_axis_name="s"),
        scratch_types=[
            pltpu.VMEM((TPW, C // 2), jnp.int32),
            pltpu.VMEM((TPW, C // 2), jnp.int32),
            pltpu.VMEM((TPW,), jnp.int32),
            pltpu.VMEM((TPW,), jnp.int32),
            pltpu.SemaphoreType.DMA,
            pltpu.SemaphoreType.DMA,
        ],
    )
    return f(yg, pos)


FRB = 512


def _finalize_body(g_ref, w_ref, out_ref):
    g0 = g_ref[0]
    g1 = g_ref[1]
    w0c = w_ref[:, 0:1]
    w1c = w_ref[:, 1:2]
    lo0 = pltpu.bitcast((g0 & 0xFFFF) << 16, jnp.float32)
    hi0 = pltpu.bitcast(g0 & -65536, jnp.float32)
    lo1 = pltpu.bitcast((g1 & 0xFFFF) << 16, jnp.float32)
    hi1 = pltpu.bitcast(g1 & -65536, jnp.float32)
    out_ref[:, 0:C // 2] = w0c * lo0 + w1c * lo1
    out_ref[:, C // 2:C] = w0c * hi0 + w1c * hi1


def _finalize(g, wcols):
    return pl.pallas_call(
        _finalize_body,
        grid=(B // FRB,),
        in_specs=[
            pl.BlockSpec((K, FRB, C // 2), lambda i: (0, i, 0)),
            pl.BlockSpec((FRB, K), lambda i: (i, 0)),
        ],
        out_specs=pl.BlockSpec((FRB, C), lambda i: (i, 0)),
        out_shape=jax.ShapeDtypeStruct((B, C), jnp.float32),
    )(g, wcols)


def kernel(x, Wg, bg, W1, b1, W2, b2):
    rout, topk, pos, wcols, emap_row, xpk = _router(x, Wg, bg.reshape(1, E))
    xg = _scatter(xpk, pos)
    yg = _moe_mm(emap_row, xg, W1, b1, W2, b2)
    g = _gather2(yg, pos)
    final = _finalize(g, wcols)
    return final, rout, topk


# speedup vs baseline: 1.1008x; 1.0257x over previous
import functools

import jax
import jax.numpy as jnp
from jax import lax
from jax.experimental import pallas as pl
from jax.experimental.pallas import tpu as pltpu
from jax.experimental.pallas import tpu_sc as plsc

B = 2048
D = 1024
E = 8
K = 2
H = 4096
C = 1024

BLK = 512
NBLK = (B * K) // BLK + E
NPAD = NBLK * BLK
NUSED_SLOT = 32

RBLK = 256
NEG_INF = float("-inf")


def _router_body(x_ref, wgt_ref, bg_ref,
                 rout_ref, topk_ref, pos_ref, w_ref, emap_ref, xpk_ref):
    x = x_ref[...]
    logits = lax.dot_general(x, wgt_ref[...], (((1,), (1,)), ((), ())),
                             preferred_element_type=jnp.float32) + bg_ref[...]

    iota_e = lax.broadcasted_iota(jnp.int32, (B, E), 1)
    v0 = jnp.max(logits, axis=1, keepdims=True)
    i0 = jnp.min(jnp.where(logits == v0, iota_e, E), axis=1, keepdims=True)
    oh0 = iota_e == i0
    masked = jnp.where(oh0, NEG_INF, logits)
    v1 = jnp.max(masked, axis=1, keepdims=True)
    i1 = jnp.min(jnp.where(masked == v1, iota_e, E), axis=1, keepdims=True)
    oh1 = iota_e == i1

    e1 = jnp.exp(v1 - v0)
    w0 = 1.0 / (1.0 + e1)
    w1 = e1 * w0
    rout_ref[...] = jnp.where(oh0, w0, 0.0) + jnp.where(oh1, w1, 0.0)
    topk_ref[...] = jnp.concatenate([i0, i1], axis=1)

    combined = oh0.astype(jnp.float32) + oh1.astype(jnp.float32)
    nblocks = B // RBLK
    base = jnp.zeros((1, E), dtype=jnp.float32)
    rank_parts = []
    tri = (lax.broadcasted_iota(jnp.int32, (RBLK, RBLK), 1)
           < lax.broadcasted_iota(jnp.int32, (RBLK, RBLK), 0)).astype(jnp.float32)
    for blk in range(nblocks):
        rows = combined[blk * RBLK:(blk + 1) * RBLK]
        rk = jnp.dot(tri, rows, preferred_element_type=jnp.float32) + base
        rank_parts.append(rk)
        base = base + jnp.sum(rows, axis=0, keepdims=True)
    rank = jnp.concatenate(rank_parts, axis=0)
    counts = base

    padded = jnp.ceil(counts / BLK) * BLK
    lt8 = (lax.broadcasted_iota(jnp.int32, (E, E), 0)
           < lax.broadcasted_iota(jnp.int32, (E, E), 1)).astype(jnp.float32)
    offs = jnp.dot(padded, lt8, preferred_element_type=jnp.float32)

    posmat = offs + rank
    pos0 = jnp.sum(jnp.where(oh0, posmat, 0.0), axis=1, keepdims=True)
    pos1 = jnp.sum(jnp.where(oh1, posmat, 0.0), axis=1, keepdims=True)
    pos_ref[0:1, :] = pos0.astype(jnp.int32).reshape(1, B)
    pos_ref[1:2, :] = pos1.astype(jnp.int32).reshape(1, B)
    w_ref[...] = jnp.concatenate([w0, w1], axis=1)

    bstart = (offs / BLK).astype(jnp.int32).reshape(E, 1)
    bidx = lax.broadcasted_iota(jnp.int32, (E, 64), 1)
    emap = (jnp.sum((bstart <= bidx).astype(jnp.int32), axis=0,
                    keepdims=True) - 1)
    nused = (jnp.sum(padded) / BLK).astype(jnp.int32)
    emap_ref[...] = jnp.where(
        lax.broadcasted_iota(jnp.int32, (1, 64), 1) == NUSED_SLOT,
        nused, emap)

    xeb = pltpu.bitcast(x[:, :D // 2].astype(jnp.bfloat16),
                        jnp.int16).astype(jnp.int32)
    xob = pltpu.bitcast(x[:, D // 2:].astype(jnp.bfloat16),
                        jnp.int16).astype(jnp.int32)
    xpk_ref[...] = (xob << 16) | (xeb & 0xFFFF)


def _router(x, wgt, bg2):
    return pl.pallas_call(
        _router_body,
        out_shape=(
            jax.ShapeDtypeStruct((B, E), jnp.float32),
            jax.ShapeDtypeStruct((B, K), jnp.int32),
            jax.ShapeDtypeStruct((K, B), jnp.int32),
            jax.ShapeDtypeStruct((B, K), jnp.float32),
            jax.ShapeDtypeStruct((1, 64), jnp.int32),
            jax.ShapeDtypeStruct((B, D // 2), jnp.int32),
        ),
    )(x, wgt, bg2)


NW = 32
TPW = B // NW


def _scatter_body(x_hbm, pos_hbm, xg_hbm, rows_v, idx0_v, idx1_v, sem0, sem1):
    wid = lax.axis_index("s") * 2 + lax.axis_index("c")
    base = wid * TPW
    pltpu.sync_copy(x_hbm.at[pl.ds(base, TPW)], rows_v)
    pltpu.sync_copy(pos_hbm.at[0, pl.ds(base, TPW)], idx0_v)
    pltpu.sync_copy(pos_hbm.at[1, pl.ds(base, TPW)], idx1_v)
    cp0 = pltpu.make_async_copy(rows_v, xg_hbm.at[idx0_v], sem0)
    cp1 = pltpu.make_async_copy(rows_v, xg_hbm.at[idx1_v], sem1)
    cp0.start()
    cp1.start()
    cp0.wait()
    cp1.wait()


def _scatter(xpk, pos):
    f = pl.kernel(
        _scatter_body,
        out_type=jax.ShapeDtypeStruct((NPAD, D // 2), jnp.int32),
        mesh=plsc.VectorSubcoreMesh(core_axis_name="c", subcore_axis_name="s"),
        scratch_types=[
            pltpu.VMEM((TPW, D // 2), jnp.int32),
            pltpu.VMEM((TPW,), jnp.int32),
            pltpu.VMEM((TPW,), jnp.int32),
            pltpu.SemaphoreType.DMA,
            pltpu.SemaphoreType.DMA,
        ],
    )
    return f(xpk, pos)


_SQRT_HALF = 0.7071067811865476


def _m1_body(emap_ref, xg_ref, w1_ref, b1_ref, h_ref):
    @pl.when(pl.program_id(0) < emap_ref[0, NUSED_SLOT])
    def _():
        xp = xg_ref[...]
        lo = pltpu.bitcast((xp & 0xFFFF) << 16, jnp.float32)
        hi = pltpu.bitcast(xp & -65536, jnp.float32)
        xb = jnp.concatenate([lo, hi], axis=1)
        pre = jnp.dot(xb, w1_ref[0],
                      preferred_element_type=jnp.float32) + b1_ref[0]
        h = 0.5 * pre * (1.0 + lax.erf(pre * _SQRT_HALF))
        h_ref[...] = h.astype(jnp.bfloat16)


def _m2_body(emap_ref, h_ref, w2_ref, b2_ref, yg_ref):
    @pl.when(pl.program_id(0) < emap_ref[0, NUSED_SLOT])
    def _():
        yg = jnp.dot(h_ref[...].astype(jnp.float32), w2_ref[0],
                     preferred_element_type=jnp.float32) + b2_ref[0]
        ye = pltpu.bitcast(yg[:, :C // 2].astype(jnp.bfloat16),
                           jnp.int16).astype(jnp.int32)
        yo = pltpu.bitcast(yg[:, C // 2:].astype(jnp.bfloat16),
                           jnp.int16).astype(jnp.int32)
        yg_ref[...] = (yo << 16) | (ye & 0xFFFF)


def _clamp(i, em):
    return jnp.minimum(i, em[0, NUSED_SLOT] - 1)


def _blk_idx(i, em):
    return (_clamp(i, em), 0)


def _w_idx(i, em):
    return (em[0, _clamp(i, em)], 0, 0)


def _moe_mm(emap, xg, W1, b1, W2, b2):
    h = pl.pallas_call(
        _m1_body,
        grid_spec=pltpu.PrefetchScalarGridSpec(
            num_scalar_prefetch=1,
            grid=(NBLK,),
            in_specs=[
                pl.BlockSpec((BLK, D // 2), _blk_idx),
                pl.BlockSpec((1, D, H), _w_idx),
                pl.BlockSpec((1, 1, H), _w_idx),
            ],
            out_specs=pl.BlockSpec((BLK, H), _blk_idx),
        ),
        out_shape=jax.ShapeDtypeStruct((NPAD, H), jnp.bfloat16),
    )(emap, xg, W1, b1.reshape(E, 1, H))
    yg = pl.pallas_call(
        _m2_body,
        grid_spec=pltpu.PrefetchScalarGridSpec(
            num_scalar_prefetch=1,
            grid=(NBLK,),
            in_specs=[
                pl.BlockSpec((BLK, H), _blk_idx),
                pl.BlockSpec((1, H, C), _w_idx),
                pl.BlockSpec((1, 1, C), _w_idx),
            ],
            out_specs=pl.BlockSpec((BLK, C // 2), _blk_idx),
        ),
        out_shape=jax.ShapeDtypeStruct((NPAD, C // 2), jnp.int32),
    )(emap, h, W2, b2.reshape(E, 1, C))
    return yg


CH = 32


def _gather2_body(yg_hbm, pos_hbm, g_hbm,
                  r0_v, r1_v, idx0_v, idx1_v, sem0, sem1):
    wid = lax.axis_index("s") * 2 + lax.axis_index("c")
    base = wid * TPW
    pltpu.sync_copy(pos_hbm.at[0, pl.ds(base, TPW)], idx0_v)
    pltpu.sync_copy(pos_hbm.at[1, pl.ds(base, TPW)], idx1_v)
    cp0 = pltpu.make_async_copy(yg_hbm.at[idx0_v], r0_v, sem0)
    cp1 = pltpu.make_async_copy(yg_hbm.at[idx1_v], r1_v, sem1)
    cp0.start()
    cp1.start()
    cp0.wait()
    pltpu.sync_copy(r0_v, g_hbm.at[0, pl.ds(base, TPW)])
    cp1.wait()
    pltpu.sync_copy(r1_v, g_hbm.at[1, pl.ds(base, TPW)])


def _gather2(yg, pos):
    f = pl.kernel(
        _gather2_body,
        out_type=jax.ShapeDtypeStruct((K, B, C // 2), jnp.int32),
        mesh=plsc.VectorSubcoreMesh(core_axis_name="c", subcore_axis_name="s"),
        scratch_types=[
            pltpu.VMEM((TPW, C // 2), jnp.int32),
            pltpu.VMEM((TPW, C // 2), jnp.int32),
            pltpu.VMEM((TPW,), jnp.int32),
            pltpu.VMEM((TPW,), jnp.int32),
            pltpu.SemaphoreType.DMA,
            pltpu.SemaphoreType.DMA,
        ],
    )
    return f(yg, pos)


FRB = 512


def _finalize_body(g_ref, w_ref, out_ref):
    g0 = g_ref[0]
    g1 = g_ref[1]
    w0c = w_ref[:, 0:1]
    w1c = w_ref[:, 1:2]
    lo0 = pltpu.bitcast((g0 & 0xFFFF) << 16, jnp.float32)
    hi0 = pltpu.bitcast(g0 & -65536, jnp.float32)
    lo1 = pltpu.bitcast((g1 & 0xFFFF) << 16, jnp.float32)
    hi1 = pltpu.bitcast(g1 & -65536, jnp.float32)
    out_ref[:, 0:C // 2] = w0c * lo0 + w1c * lo1
    out_ref[:, C // 2:C] = w0c * hi0 + w1c * hi1


def _finalize(g, wcols):
    return pl.pallas_call(
        _finalize_body,
        grid=(B // FRB,),
        in_specs=[
            pl.BlockSpec((K, FRB, C // 2), lambda i: (0, i, 0)),
            pl.BlockSpec((FRB, K), lambda i: (i, 0)),
        ],
        out_specs=pl.BlockSpec((FRB, C), lambda i: (i, 0)),
        out_shape=jax.ShapeDtypeStruct((B, C), jnp.float32),
    )(g, wcols)


def kernel(x, Wg, bg, W1, b1, W2, b2):
    rout, topk, pos, wcols, emap_row, xpk = _router(x, Wg, bg.reshape(1, E))
    xg = _scatter(xpk, pos)
    yg = _moe_mm(emap_row, xg, W1, b1, W2, b2)
    g = _gather2(yg, pos)
    final = _finalize(g, wcols)
    return final, rout, topk
